# Initial kernel scaffold; baseline (speedup 1.0000x reference)
import jax
import jax.numpy as jnp
from jax.experimental import pallas as pl


def kernel(x, som, class_count):
    raise NotImplementedError("write your pallas kernel here")


# speedup vs baseline: 2.3262x; 2.3262x over previous
import functools

import jax
import jax.numpy as jnp
from jax import lax
from jax.experimental import pallas as pl
from jax.experimental.pallas import tpu as pltpu
from jax.experimental.pallas import tpu_sc as plsc

Q = 4096
D = 128
UX = 64
UY = 64
U = UX * UY
NCLS = 10
CPAD = 128
QT = 256
GRID = Q // QT

NW = 32
BPW = Q // NW


def _tc_body(cc_ref, x_ref, w_ref, sims_ref, bmu_ref, flat_ref, pmi_ref,
             wn_ref):
    i = pl.program_id(0)

    @pl.when(i == 0)
    def _prologue():
        w = w_ref[...]
        wn_ref[...] = w / (jnp.sqrt(jnp.sum(w * w, axis=1, keepdims=True))
                           + 1e-6)
        cc = cc_ref[...]
        row_sum = jnp.sum(cc, axis=1, keepdims=True)
        cond = cc / (row_sum + 1e-6)
        prior = jnp.sum(cc, axis=0, keepdims=True) / (jnp.sum(cc) + 1e-6)
        pmi_ref[...] = jnp.log(cond / (prior + 1e-6) + 1e-6)

    x = x_ref[...]
    xn = x / (jnp.sqrt(jnp.sum(x * x, axis=1, keepdims=True)) + 1e-6)
    s = lax.dot_general(xn, wn_ref[...], (((1,), (1,)), ((), ())),
                        preferred_element_type=jnp.float32)
    sims_ref[...] = s
    idx = jnp.argmax(s, axis=1).astype(jnp.int32)
    bmu_ref[...] = jnp.stack([idx // UY, idx % UY], axis=1)
    flat_ref[0, 0, :] = idx


def _make_sc_gather():
    mesh = plsc.VectorSubcoreMesh(core_axis_name="c", subcore_axis_name="s")

    @functools.partial(
        pl.kernel,
        mesh=mesh,
        out_type=jax.ShapeDtypeStruct((Q, CPAD), jnp.float32),
        scratch_types=[
            pltpu.VMEM((BPW,), jnp.int32),
            pltpu.VMEM((BPW, CPAD), jnp.float32),
            pltpu.SemaphoreType.DMA,
        ],
    )
    def _sc_gather(pmi_hbm, idx_hbm, out_hbm, idx_v, rows_v, sem):
        wid = lax.axis_index("s") * 2 + lax.axis_index("c")
        base = wid * BPW
        pltpu.sync_copy(idx_hbm.at[pl.ds(base, BPW)], idx_v)
        pltpu.async_copy(pmi_hbm.at[idx_v], rows_v, sem).wait()
        pltpu.sync_copy(rows_v, out_hbm.at[pl.ds(base, BPW)])

    return _sc_gather


def kernel(x, som, class_count):
    w = som.reshape(U, D)
    cc = class_count.reshape(U, NCLS)
    cc_pad = jnp.concatenate(
        [cc, jnp.zeros((U, CPAD - NCLS), cc.dtype)], axis=1)

    sims, bmu, flat3, pmi_pad = pl.pallas_call(
        _tc_body,
        grid=(GRID,),
        in_specs=[
            pl.BlockSpec((U, CPAD), lambda i: (0, 0)),
            pl.BlockSpec((QT, D), lambda i: (i, 0)),
            pl.BlockSpec((U, D), lambda i: (0, 0)),
        ],
        out_specs=[
            pl.BlockSpec((QT, U), lambda i: (i, 0)),
            pl.BlockSpec((QT, 2), lambda i: (i, 0)),
            pl.BlockSpec((1, 1, QT), lambda i: (i, 0, 0)),
            pl.BlockSpec((U, CPAD), lambda i: (0, 0)),
        ],
        out_shape=[
            jax.ShapeDtypeStruct((Q, U), jnp.float32),
            jax.ShapeDtypeStruct((Q, 2), jnp.int32),
            jax.ShapeDtypeStruct((GRID, 1, QT), jnp.int32),
            jax.ShapeDtypeStruct((U, CPAD), jnp.float32),
        ],
        scratch_shapes=[pltpu.VMEM((U, D), jnp.float32)],
    )(cc_pad, x, w)

    bmu_flat = flat3.reshape(Q)
    bmu_pmi_pad = _make_sc_gather()(pmi_pad, bmu_flat)
    return sims, bmu, bmu_pmi_pad[:, :NCLS]


# speedup vs baseline: 2.4417x; 1.0496x over previous
import functools

import jax
import jax.numpy as jnp
from jax import lax
from jax.experimental import pallas as pl
from jax.experimental.pallas import tpu as pltpu
from jax.experimental.pallas import tpu_sc as plsc

Q = 4096
D = 128
UX = 64
UY = 64
U = UX * UY
NCLS = 10
CPAD = 128
QT = 512
GRID = Q // QT

NW = 32
BPW = Q // NW


def _tc_body(cc_ref, x_ref, w_ref, sims_ref, bmu_ref, flat_ref, pmi_ref,
             wn_ref):
    i = pl.program_id(0)

    @pl.when(i == 0)
    def _prologue():
        w = w_ref[...]
        wn_ref[...] = w / (jnp.sqrt(jnp.sum(w * w, axis=1, keepdims=True))
                           + 1e-6)
        cc = jnp.concatenate(
            [cc_ref[...], jnp.zeros((U, CPAD - NCLS), jnp.float32)], axis=1)
        row_sum = jnp.sum(cc, axis=1, keepdims=True)
        cond = cc / (row_sum + 1e-6)
        prior = jnp.sum(cc, axis=0, keepdims=True) / (jnp.sum(cc) + 1e-6)
        pmi_ref[...] = jnp.log(cond / (prior + 1e-6) + 1e-6)

    x = x_ref[...]
    xn = x / (jnp.sqrt(jnp.sum(x * x, axis=1, keepdims=True)) + 1e-6)
    s = lax.dot_general(xn, wn_ref[...], (((1,), (1,)), ((), ())),
                        preferred_element_type=jnp.float32)
    sims_ref[...] = s
    idx = jnp.argmax(s, axis=1).astype(jnp.int32)
    bmu_ref[...] = jnp.stack([idx // UY, idx % UY], axis=1)
    flat_ref[...] = idx


def _make_sc_gather():
    mesh = plsc.VectorSubcoreMesh(core_axis_name="c", subcore_axis_name="s")

    @functools.partial(
        pl.kernel,
        mesh=mesh,
        out_type=jax.ShapeDtypeStruct((Q, CPAD), jnp.float32),
        scratch_types=[
            pltpu.VMEM((BPW,), jnp.int32),
            pltpu.VMEM((BPW, CPAD), jnp.float32),
            pltpu.SemaphoreType.DMA,
        ],
    )
    def _sc_gather(pmi_hbm, idx_hbm, out_hbm, idx_v, rows_v, sem):
        wid = lax.axis_index("s") * 2 + lax.axis_index("c")
        base = wid * BPW
        pltpu.sync_copy(idx_hbm.at[pl.ds(base, BPW)], idx_v)
        pltpu.async_copy(pmi_hbm.at[idx_v], rows_v, sem).wait()
        pltpu.sync_copy(rows_v, out_hbm.at[pl.ds(base, BPW)])

    return _sc_gather


def kernel(x, som, class_count):
    w = som.reshape(U, D)
    cc = class_count.reshape(U, NCLS)

    sims, bmu, bmu_flat, pmi_pad = pl.pallas_call(
        _tc_body,
        grid=(GRID,),
        in_specs=[
            pl.BlockSpec((U, NCLS), lambda i: (0, 0)),
            pl.BlockSpec((QT, D), lambda i: (i, 0)),
            pl.BlockSpec((U, D), lambda i: (0, 0)),
        ],
        out_specs=[
            pl.BlockSpec((QT, U), lambda i: (i, 0)),
            pl.BlockSpec((QT, 2), lambda i: (i, 0)),
            pl.BlockSpec((QT,), lambda i: (i,)),
            pl.BlockSpec((U, CPAD), lambda i: (0, 0)),
        ],
        out_shape=[
            jax.ShapeDtypeStruct((Q, U), jnp.float32),
            jax.ShapeDtypeStruct((Q, 2), jnp.int32),
            jax.ShapeDtypeStruct((Q,), jnp.int32),
            jax.ShapeDtypeStruct((U, CPAD), jnp.float32),
        ],
        scratch_shapes=[pltpu.VMEM((U, D), jnp.float32)],
    )(cc, x, w)

    bmu_pmi_pad = _make_sc_gather()(pmi_pad, bmu_flat)
    return sims, bmu, bmu_pmi_pad[:, :NCLS]
